# baseline (speedup 1.0000x reference)
import functools

import jax
import jax.numpy as jnp
from jax import lax
from jax.experimental import pallas as pl
from jax.experimental.pallas import tpu as pltpu
from jax.experimental.pallas import tpu_sc as plsc

B, L, D, C = 4096, 50, 64, 4
V = 1000000
NC, NS = 2, 16
NW = NC * NS
BPW = B // NW
CH = 2
NIDX = 104
NCHUNK = BPW // CH
LANES = 16
PADCH = 112

RB = 8192


def _gt_body(w_ref, t_ref, o_ref):
    o_ref[...] = lax.dot_general(
        w_ref[...], t_ref[...], (((1,), (1,)), ((), ())),
        precision=lax.Precision.HIGHEST, preferred_element_type=jnp.float32)


_gt_matmul = pl.pallas_call(
    _gt_body,
    grid=(pl.cdiv(V, RB),),
    in_specs=[
        pl.BlockSpec((C, D), lambda i: (0, 0)),
        pl.BlockSpec((RB, D), lambda i: (i, 0)),
    ],
    out_specs=pl.BlockSpec((C, RB), lambda i: (0, i)),
    out_shape=jax.ShapeDtypeStruct((C, V), jnp.float32),
)

_mesh = plsc.VectorSubcoreMesh(
    core_axis_name="c", subcore_axis_name="s", num_cores=NC, num_subcores=NS)


_pool_scratch = [
    pltpu.VMEM((NCHUNK, NIDX), jnp.int32),
    pltpu.VMEM((2, C, PADCH), jnp.float32),
    pltpu.VMEM((BPW, C * LANES), jnp.float32),
    pltpu.SemaphoreType.DMA,
    pltpu.SemaphoreType.DMA,
]


def _pool_sum_body(text_hbm, g0_hbm, g1_hbm, g2_hbm, g3_hbm, out_hbm,
                   idx_v, gbuf, part_v, sem0, sem1):
    wid = lax.axis_index("s") * NC + lax.axis_index("c")
    pltpu.sync_copy(text_hbm.at[pl.ds(wid * NCHUNK, NCHUNK)], idx_v)
    gs = (g0_hbm, g1_hbm, g2_hbm, g3_hbm)
    sems = (sem0, sem1)
    lane = lax.iota(jnp.int32, LANES)
    m_lo2 = lane < 2
    m_hi2 = lane >= 2
    m_lo4 = lane < 4

    def start(c):
        buf = c % 2
        return [
            pltpu.async_copy(gs[k].at[idx_v.at[c]],
                             gbuf.at[buf, k, pl.ds(0, NIDX)], sems[buf])
            for k in range(C)
        ]

    pending = start(0)
    for c in range(NCHUNK):
        nxt = start(c + 1) if c + 1 < NCHUNK else None
        for h in pending:
            h.wait()
        buf = c % 2
        for k in range(C):
            v = [gbuf[buf, k, pl.ds(q * LANES, LANES)] for q in range(7)]
            zero = jnp.zeros((LANES,), jnp.float32)
            bag0 = v[0] + v[1] + v[2] + jnp.where(m_lo2, v[3], zero)
            bag1 = (jnp.where(m_hi2, v[3], zero) + v[4] + v[5]
                    + jnp.where(m_lo4, v[6], zero))
            part_v[CH * c + 0, pl.ds(k * LANES, LANES)] = bag0
            part_v[CH * c + 1, pl.ds(k * LANES, LANES)] = bag1
        pending = nxt
    pltpu.sync_copy(part_v, out_hbm.at[pl.ds(wid * BPW, BPW)])


_pool_sum = pl.kernel(
    _pool_sum_body,
    out_type=jax.ShapeDtypeStruct((B, C * LANES), jnp.float32),
    mesh=_mesh,
    scratch_types=_pool_scratch,
    compiler_params=pltpu.CompilerParams(use_tc_tiling_on_sc=False),
)


def _fold_body(p_ref, s_ref, b_ref, o_ref):
    o_ref[...] = lax.dot_general(
        p_ref[...], s_ref[...], (((1,), (1,)), ((), ())),
        precision=lax.Precision.HIGHEST,
        preferred_element_type=jnp.float32) + b_ref[...]


_fold = pl.pallas_call(
    _fold_body,
    out_shape=jax.ShapeDtypeStruct((B, C), jnp.float32),
)


def kernel(text, table, Wfc, bfc):
    gt = _gt_matmul(Wfc, table)
    g0, g1, g2, g3 = gt[0], gt[1], gt[2], gt[3]
    text2 = jnp.pad(text.reshape(NW * NCHUNK, CH * L),
                    ((0, 0), (0, NIDX - CH * L)))
    part = _pool_sum(text2, g0, g1, g2, g3)
    sel = jnp.repeat(jnp.eye(C, dtype=jnp.float32), LANES, axis=1) / L
    return _fold(part, sel, bfc.reshape(1, C))


# speedup vs baseline: 1.4746x; 1.4746x over previous
import functools

import jax
import jax.numpy as jnp
from jax import lax
from jax.experimental import pallas as pl
from jax.experimental.pallas import tpu as pltpu
from jax.experimental.pallas import tpu_sc as plsc

B, L, D, C = 4096, 50, 64, 4
V = 1000000
NC, NS = 2, 16
NW = NC * NS
BPW = B // NW
CH = 2
NIDX = 104
NCHUNK = BPW // CH
LANES = 16
PADCH = 112

RB = 8192


def _gt_body(w_ref, t_ref, o0_ref, o1_ref, o2_ref, o3_ref):
    gt = lax.dot_general(
        w_ref[...], t_ref[...], (((1,), (1,)), ((), ())),
        preferred_element_type=jnp.float32)
    o0_ref[...] = gt[0]
    o1_ref[...] = gt[1]
    o2_ref[...] = gt[2]
    o3_ref[...] = gt[3]


_gt_matmul = pl.pallas_call(
    _gt_body,
    grid=(pl.cdiv(V, RB),),
    in_specs=[
        pl.BlockSpec((C, D), lambda i: (0, 0)),
        pl.BlockSpec((RB, D), lambda i: (i, 0)),
    ],
    out_specs=[pl.BlockSpec((RB,), lambda i: (i,)) for _ in range(C)],
    out_shape=[jax.ShapeDtypeStruct((V,), jnp.float32) for _ in range(C)],
)

_mesh = plsc.VectorSubcoreMesh(
    core_axis_name="c", subcore_axis_name="s", num_cores=NC, num_subcores=NS)


_pool_scratch = [
    pltpu.VMEM((NCHUNK, NIDX), jnp.int32),
    pltpu.VMEM((2, C, PADCH), jnp.float32),
    pltpu.VMEM((BPW, C * LANES), jnp.float32),
    pltpu.SemaphoreType.DMA,
    pltpu.SemaphoreType.DMA,
]


def _pool_sum_body(text_hbm, g0_hbm, g1_hbm, g2_hbm, g3_hbm, out_hbm,
                   idx_v, gbuf, part_v, sem0, sem1):
    wid = lax.axis_index("s") * NC + lax.axis_index("c")
    pltpu.sync_copy(text_hbm.at[pl.ds(wid * NCHUNK, NCHUNK)], idx_v)
    gs = (g0_hbm, g1_hbm, g2_hbm, g3_hbm)
    sems = (sem0, sem1)
    lane = lax.iota(jnp.int32, LANES)
    m_lo2 = lane < 2
    m_hi2 = lane >= 2
    m_lo4 = lane < 4

    def start(c):
        buf = c % 2
        return [
            pltpu.async_copy(gs[k].at[idx_v.at[c]],
                             gbuf.at[buf, k, pl.ds(0, NIDX)], sems[buf])
            for k in range(C)
        ]

    pending = start(0)
    for c in range(NCHUNK):
        nxt = start(c + 1) if c + 1 < NCHUNK else None
        for h in pending:
            h.wait()
        buf = c % 2
        for k in range(C):
            v = [gbuf[buf, k, pl.ds(q * LANES, LANES)] for q in range(7)]
            zero = jnp.zeros((LANES,), jnp.float32)
            bag0 = v[0] + v[1] + v[2] + jnp.where(m_lo2, v[3], zero)
            bag1 = (jnp.where(m_hi2, v[3], zero) + v[4] + v[5]
                    + jnp.where(m_lo4, v[6], zero))
            part_v[CH * c + 0, pl.ds(k * LANES, LANES)] = bag0
            part_v[CH * c + 1, pl.ds(k * LANES, LANES)] = bag1
        pending = nxt
    pltpu.sync_copy(part_v, out_hbm.at[pl.ds(wid * BPW, BPW)])


_pool_sum = pl.kernel(
    _pool_sum_body,
    out_type=jax.ShapeDtypeStruct((B, C * LANES), jnp.float32),
    mesh=_mesh,
    scratch_types=_pool_scratch,
    compiler_params=pltpu.CompilerParams(use_tc_tiling_on_sc=False),
)


def _fold_body(p_ref, s_ref, b_ref, o_ref):
    o_ref[...] = lax.dot_general(
        p_ref[...], s_ref[...], (((1,), (1,)), ((), ())),
        precision=lax.Precision.HIGHEST,
        preferred_element_type=jnp.float32) + b_ref[...]


_fold = pl.pallas_call(
    _fold_body,
    out_shape=jax.ShapeDtypeStruct((B, C), jnp.float32),
)


def kernel(text, table, Wfc, bfc):
    g0, g1, g2, g3 = _gt_matmul(Wfc, table)
    text2 = jnp.pad(text.reshape(NW * NCHUNK, CH * L),
                    ((0, 0), (0, NIDX - CH * L)))
    part = _pool_sum(text2, g0, g1, g2, g3)
    sel = jnp.repeat(jnp.eye(C, dtype=jnp.float32), LANES, axis=1) / L
    return _fold(part, sel, bfc.reshape(1, C))


# speedup vs baseline: 4.2168x; 2.8596x over previous
import functools

import jax
import jax.numpy as jnp
from jax import lax
from jax.experimental import pallas as pl
from jax.experimental.pallas import tpu as pltpu
from jax.experimental.pallas import tpu_sc as plsc

B, L, D, C = 4096, 50, 64, 4
V = 1000000
NC, NS = 2, 16
NW = NC * NS
BPW = B // NW
CH = 2
NIDX = 104
NCHUNK = BPW // CH
LANES = 16
PADCH = 112

RB = 8192


def _gt_body(w_ref, t_ref, o0_ref, o1_ref, o2_ref, o3_ref):
    gt = lax.dot_general(
        w_ref[...], t_ref[...], (((1,), (0,)), ((), ())),
        preferred_element_type=jnp.float32)
    o0_ref[...] = gt[0]
    o1_ref[...] = gt[1]
    o2_ref[...] = gt[2]
    o3_ref[...] = gt[3]


_gt_matmul = pl.pallas_call(
    _gt_body,
    grid=(pl.cdiv(V, RB),),
    in_specs=[
        pl.BlockSpec((C, D), lambda i: (0, 0)),
        pl.BlockSpec((D, RB), lambda i: (0, i)),
    ],
    out_specs=[pl.BlockSpec((RB,), lambda i: (i,)) for _ in range(C)],
    out_shape=[jax.ShapeDtypeStruct((V,), jnp.float32) for _ in range(C)],
)

_mesh = plsc.VectorSubcoreMesh(
    core_axis_name="c", subcore_axis_name="s", num_cores=NC, num_subcores=NS)


_pool_scratch = [
    pltpu.VMEM((NCHUNK, NIDX), jnp.int32),
    pltpu.VMEM((2, C, PADCH), jnp.float32),
    pltpu.VMEM((BPW, C * LANES), jnp.float32),
    pltpu.SemaphoreType.DMA,
    pltpu.SemaphoreType.DMA,
]


def _pool_sum_body(text_hbm, g0_hbm, g1_hbm, g2_hbm, g3_hbm, out_hbm,
                   idx_v, gbuf, part_v, sem0, sem1):
    wid = lax.axis_index("s") * NC + lax.axis_index("c")
    pltpu.sync_copy(text_hbm.at[pl.ds(wid * NCHUNK, NCHUNK)], idx_v)
    gs = (g0_hbm, g1_hbm, g2_hbm, g3_hbm)
    sems = (sem0, sem1)
    lane = lax.iota(jnp.int32, LANES)
    m_lo2 = lane < 2
    m_hi2 = lane >= 2
    m_lo4 = lane < 4

    def start(c):
        buf = c % 2
        return [
            pltpu.async_copy(gs[k].at[idx_v.at[c]],
                             gbuf.at[buf, k, pl.ds(0, NIDX)], sems[buf])
            for k in range(C)
        ]

    pending = start(0)
    for c in range(NCHUNK):
        nxt = start(c + 1) if c + 1 < NCHUNK else None
        for h in pending:
            h.wait()
        buf = c % 2
        for k in range(C):
            v = [gbuf[buf, k, pl.ds(q * LANES, LANES)] for q in range(7)]
            zero = jnp.zeros((LANES,), jnp.float32)
            bag0 = v[0] + v[1] + v[2] + jnp.where(m_lo2, v[3], zero)
            bag1 = (jnp.where(m_hi2, v[3], zero) + v[4] + v[5]
                    + jnp.where(m_lo4, v[6], zero))
            part_v[CH * c + 0, pl.ds(k * LANES, LANES)] = bag0
            part_v[CH * c + 1, pl.ds(k * LANES, LANES)] = bag1
        pending = nxt
    pltpu.sync_copy(part_v, out_hbm.at[pl.ds(wid * BPW, BPW)])


_pool_sum = pl.kernel(
    _pool_sum_body,
    out_type=jax.ShapeDtypeStruct((B, C * LANES), jnp.float32),
    mesh=_mesh,
    scratch_types=_pool_scratch,
    compiler_params=pltpu.CompilerParams(use_tc_tiling_on_sc=False),
)


def _fold_body(p_ref, s_ref, b_ref, o_ref):
    o_ref[...] = lax.dot_general(
        p_ref[...], s_ref[...], (((1,), (1,)), ((), ())),
        precision=lax.Precision.HIGHEST,
        preferred_element_type=jnp.float32) + b_ref[...]


_fold = pl.pallas_call(
    _fold_body,
    out_shape=jax.ShapeDtypeStruct((B, C), jnp.float32),
)


def kernel(text, table, Wfc, bfc):
    g0, g1, g2, g3 = _gt_matmul(Wfc, table.T)
    text2 = jnp.pad(text.reshape(NW * NCHUNK, CH * L),
                    ((0, 0), (0, NIDX - CH * L)))
    part = _pool_sum(text2, g0, g1, g2, g3)
    sel = jnp.repeat(jnp.eye(C, dtype=jnp.float32), LANES, axis=1) / L
    return _fold(part, sel, bfc.reshape(1, C))


# speedup vs baseline: 5.3145x; 1.2603x over previous
import functools

import jax
import jax.numpy as jnp
from jax import lax
from jax.experimental import pallas as pl
from jax.experimental.pallas import tpu as pltpu
from jax.experimental.pallas import tpu_sc as plsc

B, L, D, C = 4096, 50, 64, 4
V = 1000000
NC, NS = 2, 16
NW = NC * NS
BPW = B // NW
CH = 2
NIDX = 104
NCHUNK = BPW // CH
LANES = 16
PADCH = 112

RB = 32768


def _gt_body(w_ref, t_ref, o0_ref, o1_ref, o2_ref, o3_ref):
    gt = lax.dot_general(
        w_ref[...], t_ref[...], (((1,), (0,)), ((), ())),
        preferred_element_type=jnp.float32)
    o0_ref[...] = gt[0]
    o1_ref[...] = gt[1]
    o2_ref[...] = gt[2]
    o3_ref[...] = gt[3]


_gt_matmul = pl.pallas_call(
    _gt_body,
    grid=(pl.cdiv(V, RB),),
    in_specs=[
        pl.BlockSpec((C, D), lambda i: (0, 0)),
        pl.BlockSpec((D, RB), lambda i: (0, i)),
    ],
    out_specs=[pl.BlockSpec((RB,), lambda i: (i,)) for _ in range(C)],
    out_shape=[jax.ShapeDtypeStruct((V,), jnp.float32) for _ in range(C)],
)

_mesh = plsc.VectorSubcoreMesh(
    core_axis_name="c", subcore_axis_name="s", num_cores=NC, num_subcores=NS)


NBUF = 8

_pool_scratch = [
    pltpu.VMEM((NCHUNK, NIDX), jnp.int32),
    pltpu.VMEM((NBUF, C, PADCH), jnp.float32),
    pltpu.VMEM((BPW, C * LANES), jnp.float32),
] + [pltpu.SemaphoreType.DMA] * NBUF


def _pool_sum_body(text_hbm, g0_hbm, g1_hbm, g2_hbm, g3_hbm, out_hbm,
                   idx_v, gbuf, part_v, *sems):
    wid = lax.axis_index("s") * NC + lax.axis_index("c")
    pltpu.sync_copy(text_hbm.at[pl.ds(wid * NCHUNK, NCHUNK)], idx_v)
    gs = (g0_hbm, g1_hbm, g2_hbm, g3_hbm)
    lane = lax.iota(jnp.int32, LANES)
    m_lo2 = lane < 2
    m_hi2 = lane >= 2
    m_lo4 = lane < 4

    def start(c):
        slot = c % NBUF
        return [
            pltpu.async_copy(gs[k].at[idx_v.at[c]],
                             gbuf.at[slot, k, pl.ds(0, NIDX)], sems[slot])
            for k in range(C)
        ]

    pending = [start(c) for c in range(NBUF - 1)]
    for c in range(NCHUNK):
        if c + NBUF - 1 < NCHUNK:
            pending.append(start(c + NBUF - 1))
        for h in pending.pop(0):
            h.wait()
        slot = c % NBUF
        for k in range(C):
            v = [gbuf[slot, k, pl.ds(q * LANES, LANES)] for q in range(7)]
            zero = jnp.zeros((LANES,), jnp.float32)
            bag0 = v[0] + v[1] + v[2] + jnp.where(m_lo2, v[3], zero)
            bag1 = (jnp.where(m_hi2, v[3], zero) + v[4] + v[5]
                    + jnp.where(m_lo4, v[6], zero))
            part_v[CH * c + 0, pl.ds(k * LANES, LANES)] = bag0
            part_v[CH * c + 1, pl.ds(k * LANES, LANES)] = bag1
    pltpu.sync_copy(part_v, out_hbm.at[pl.ds(wid * BPW, BPW)])


_pool_sum = pl.kernel(
    _pool_sum_body,
    out_type=jax.ShapeDtypeStruct((B, C * LANES), jnp.float32),
    mesh=_mesh,
    scratch_types=_pool_scratch,
    compiler_params=pltpu.CompilerParams(use_tc_tiling_on_sc=False),
)


def _fold_body(p_ref, s_ref, b_ref, o_ref):
    o_ref[...] = lax.dot_general(
        p_ref[...], s_ref[...], (((1,), (1,)), ((), ())),
        precision=lax.Precision.HIGHEST,
        preferred_element_type=jnp.float32) + b_ref[...]


_fold = pl.pallas_call(
    _fold_body,
    out_shape=jax.ShapeDtypeStruct((B, C), jnp.float32),
)


def kernel(text, table, Wfc, bfc):
    g0, g1, g2, g3 = _gt_matmul(Wfc, table.T)
    text2 = jnp.pad(text.reshape(NW * NCHUNK, CH * L),
                    ((0, 0), (0, NIDX - CH * L)))
    part = _pool_sum(text2, g0, g1, g2, g3)
    sel = jnp.repeat(jnp.eye(C, dtype=jnp.float32), LANES, axis=1) / L
    return _fold(part, sel, bfc.reshape(1, C))
